# baseline (speedup 1.0000x reference)
import jax
import jax.numpy as jnp
from jax import lax
from jax.experimental import pallas as pl
from jax.experimental.pallas import tpu as pltpu
from jax.experimental.pallas import tpu_sc as plsc

NUM_CORES = 2
L_TASKS = 200
BL = 128


def _body(iv_hbm, wt_hbm, ov_hbm, idxall, pairidx, par64, gbuf, tb,
          semg, semo, semi):
    wid = lax.axis_index("s") * NUM_CORES + lax.axis_index("c")
    iota16 = lax.iota(jnp.int32, 16)

    pltpu.async_copy(iv_hbm.at[:, wid], idxall, semi).wait()

    def tec_idx(l, b):
        lt = l // 8
        ls = l % 8
        for k in range(8):
            v = idxall[lt, ls, pl.ds(16 * k, 16)]
            pairidx[b][pl.ds(16 * k, 16)] = lax.shift_right_logical(v, 1)
            par64[b][pl.ds(16 * k, 16)] = lax.shift_left(
                lax.bitwise_and(v, 1), 6)

    def start_gather(b):
        pltpu.async_copy(wt_hbm.at[pairidx[b]], gbuf[b], semg[b])

    def wait_gather(b):
        pltpu.make_async_copy(wt_hbm.at[pairidx[b]], gbuf[b],
                              semg[b]).wait()

    def start_out(l, b):
        for ho in range(8):
            pltpu.async_copy(tb[b].at[pl.ds(8 * ho, 8), pl.ds(0, 128)],
                             ov_hbm.at[l, ho, wid], semo[b])

    def wait_out(l, b):
        for ho in range(8):
            pltpu.make_async_copy(tb[b].at[pl.ds(8 * ho, 8), pl.ds(0, 128)],
                                  ov_hbm.at[l, ho, wid], semo[b]).wait()

    h_idx = [16 * k + iota16 for k in range(4)]

    def transpose_task(b):
        def group_body(j, carry):
            for i in range(16):
                bl = 16 * j + i
                blv = jnp.full((16,), bl, jnp.int32)
                c0v = plsc.load_gather(par64[b], [blv])
                for k in range(4):
                    vals = plsc.load_gather(gbuf[b],
                                            [blv, c0v + (16 * k + iota16)])
                    plsc.store_scatter(tb[b], [h_idx[k], blv], vals)
            return carry

        lax.fori_loop(0, BL // 16, group_body, 0)

    tec_idx(0, 0)
    start_gather(0)
    tec_idx(1, 1)
    start_gather(1)

    def outer(j, carry):
        for b in range(2):
            l = 2 * j + b
            wait_gather(b)

            @pl.when(l >= 2)
            def _():
                wait_out(l - 2, b)

            transpose_task(b)

            @pl.when(l + 2 < L_TASKS)
            def _():
                tec_idx(l + 2, b)
                start_gather(b)

            start_out(l, b)
        return carry

    lax.fori_loop(0, L_TASKS // 2, outer, 0)
    wait_out(L_TASKS - 2, 0)
    wait_out(L_TASKS - 1, 1)


def kernel(inputs, weight):
    iv = inputs.T.reshape(25, 8, 32, 128).transpose(0, 2, 1, 3)
    wt = weight.reshape(500000, 128)
    mesh = plsc.VectorSubcoreMesh(core_axis_name="c", subcore_axis_name="s")
    k = pl.kernel(
        _body,
        mesh=mesh,
        out_type=jax.ShapeDtypeStruct((200, 8, 32, 8, 128), jnp.float32),
        scratch_types=[
            pltpu.VMEM((25, 8, 128), jnp.int32),
            [pltpu.VMEM((128,), jnp.int32) for _ in range(2)],
            [pltpu.VMEM((128,), jnp.int32) for _ in range(2)],
            [pltpu.VMEM((128, 128), jnp.float32) for _ in range(2)],
            [pltpu.VMEM((64, 129), jnp.float32) for _ in range(2)],
            [pltpu.SemaphoreType.DMA for _ in range(2)],
            [pltpu.SemaphoreType.DMA for _ in range(2)],
            pltpu.SemaphoreType.DMA,
        ],
        compiler_params=pltpu.CompilerParams(needs_layout_passes=False),
    )
    ov = k(iv, wt)
    return ov.transpose(2, 4, 0, 1, 3).reshape(4096, 200, 64)


# speedup vs baseline: 1.5025x; 1.5025x over previous
import jax
import jax.numpy as jnp
from jax import lax
from jax.experimental import pallas as pl
from jax.experimental.pallas import tpu as pltpu
from jax.experimental.pallas import tpu_sc as plsc

HIDDEN = 64
NUM_CORES = 2
NUM_SUBCORES = 16
NUM_WORKERS = NUM_CORES * NUM_SUBCORES
CHUNK = 400
NBUF = 4


def _gather_body(idx_hbm, table_hbm, out_hbm, *refs):
    idx_bufs = refs[0:NBUF]
    row_bufs = refs[NBUF:2 * NBUF]
    g_sems = refs[2 * NBUF:3 * NBUF]
    o_sems = refs[3 * NBUF:4 * NBUF]

    wid = lax.axis_index("s") * NUM_CORES + lax.axis_index("c")
    b_per_w = idx_hbm.shape[0] // NUM_WORKERS
    n_chunks = b_per_w // NBUF // CHUNK * NBUF
    w_base = wid * b_per_w

    def start_gather(c, b):
        pltpu.sync_copy(idx_hbm.at[pl.ds(w_base + c * CHUNK, CHUNK)],
                        idx_bufs[b])
        pltpu.async_copy(table_hbm.at[idx_bufs[b]], row_bufs[b], g_sems[b])

    def wait_gather(b):
        pltpu.make_async_copy(table_hbm.at[idx_bufs[b]], row_bufs[b],
                              g_sems[b]).wait()

    def start_write(c, b):
        pltpu.async_copy(row_bufs[b],
                         out_hbm.at[pl.ds(w_base + c * CHUNK, CHUNK)],
                         o_sems[b])

    def wait_write(c, b):
        pltpu.make_async_copy(row_bufs[b],
                              out_hbm.at[pl.ds(w_base + c * CHUNK, CHUNK)],
                              o_sems[b]).wait()

    start_gather(0, 0)
    start_gather(1, 1)

    def outer(j, carry):
        for b in range(NBUF):
            c = j * NBUF + b
            b2 = (b + 2) % NBUF

            @pl.when(c + 2 < n_chunks)
            def _prefetch():
                @pl.when(c >= 2)
                def _drain():
                    wait_write(c - 2, b2)
                start_gather(c + 2, b2)

            wait_gather(b)
            start_write(c, b)
        return carry

    lax.fori_loop(0, n_chunks // NBUF, outer, 0)

    for b in range(NBUF):
        wait_write(n_chunks - NBUF + b, b)


def kernel(inputs, weight):
    batch, length = inputs.shape
    total = batch * length
    flat_idx = inputs.reshape(total).astype(jnp.int32)
    mesh = plsc.VectorSubcoreMesh(core_axis_name="c", subcore_axis_name="s")
    scratch = ([pltpu.VMEM((CHUNK,), jnp.int32) for _ in range(NBUF)]
               + [pltpu.VMEM((CHUNK, HIDDEN), jnp.float32) for _ in range(NBUF)]
               + [pltpu.SemaphoreType.DMA for _ in range(2 * NBUF)])
    k = pl.kernel(
        _gather_body,
        mesh=mesh,
        out_type=jax.ShapeDtypeStruct((total, HIDDEN), jnp.float32),
        scratch_types=scratch,
        compiler_params=pltpu.CompilerParams(use_tc_tiling_on_sc=False),
    )
    out = k(flat_idx, weight)
    return out.reshape(batch, length, HIDDEN)
